# baseline (speedup 1.0000x reference)
import jax
import jax.numpy as jnp
from jax.experimental import pallas as pl

T = 2048
D = 1024
H = 16
KVH = 4
DH = 64
RD = 32
E = 8
I = 512
EPS = 1e-6
THETA = 10000.0
RSF = 1.0

BT = 256
BTP = 512
BQ = 512


def _rope_mats(cos, sin, nrow):
    z = jnp.zeros((nrow, DH - RD), jnp.float32)
    a = jnp.concatenate([cos, cos, z + 1.0], axis=1)
    b = jnp.concatenate([sin, sin, z], axis=1)
    r = jax.lax.broadcasted_iota(jnp.int32, (DH, DH), 0)
    c = jax.lax.broadcasted_iota(jnp.int32, (DH, DH), 1)
    p = jnp.where(jnp.logical_and(c == r + (RD // 2), r < RD // 2), 1.0, 0.0) \
        + jnp.where(jnp.logical_and(c == r - (RD // 2), r < RD), -1.0, 0.0)
    return a, b, p


def _row_sum(x):
    n = x.shape[-1]
    acc = x[:, 0:128]
    for i in range(1, n // 128):
        acc = acc + x[:, i * 128:(i + 1) * 128]
    w = acc.shape[-1] // 2
    while w >= 1:
        acc = acc[:, :w] + acc[:, w:2 * w]
        w //= 2
    return acc


def _rope_apply(x, a, b, p):
    rot = jnp.dot(x, p, preferred_element_type=jnp.float32,
                  precision=jax.lax.Precision.HIGHEST)
    return x * a + rot * b


def _qkv_body(h_ref, wq_ref, wk_ref, wv_ref, q_ref, k_ref, v_ref):
    hb = h_ref[...]
    q_ref[...] = jnp.dot(hb, wq_ref[...], preferred_element_type=jnp.float32)
    k_ref[...] = jnp.dot(hb, wk_ref[...], preferred_element_type=jnp.float32)
    v_ref[...] = jnp.dot(hb, wv_ref[...], preferred_element_type=jnp.float32)


def _attn_body(q_ref, k_ref, v_ref, cq_ref, sq_ref, ck_ref, sk_ref, o_ref):
    qi = pl.program_id(1)
    scale = DH ** -0.5
    aq, bq, p = _rope_mats(cq_ref[...], sq_ref[...], BQ)
    ak, bk, _ = _rope_mats(ck_ref[...], sk_ref[...], T)
    k = _rope_apply(k_ref[0], ak, bk, p)
    v = v_ref[0]
    row = qi * BQ + jax.lax.broadcasted_iota(jnp.int32, (BQ, T), 0)
    col = jax.lax.broadcasted_iota(jnp.int32, (BQ, T), 1)
    mask = col <= row
    outs = []
    for j in range(2):
        q = _rope_apply(q_ref[j], aq, bq, p)
        s = jax.lax.dot_general(q, k, (((1,), (1,)), ((), ())),
                                preferred_element_type=jnp.float32) * scale
        s = jnp.where(mask, s, -1e30)
        m = jnp.max(s, axis=-1, keepdims=True)
        pr = jnp.exp(s - m)
        pr = pr / _row_sum(pr)
        outs.append(jnp.dot(pr, v, preferred_element_type=jnp.float32))
    o_ref[...] = jnp.concatenate(outs, axis=1)


def _postattn_body(attn_ref, wo_ref, res_ref, res2_ref):
    o = jnp.dot(attn_ref[...], wo_ref[...], preferred_element_type=jnp.float32)
    res2_ref[...] = o + res_ref[...]


def _router_body(h2_ref, gwt_ref, eb_ref, cw_ref):
    logits = jnp.dot(h2_ref[...], gwt_ref[...],
                     preferred_element_type=jnp.float32)
    sig = jax.nn.sigmoid(logits)
    biased = sig + eb_ref[...]
    lane = jax.lax.broadcasted_iota(jnp.int32, biased.shape, 1)
    m1 = jnp.max(biased, axis=-1, keepdims=True)
    idx1 = jnp.min(jnp.where(biased == m1, lane, E), axis=-1, keepdims=True)
    sel1 = lane == idx1
    w1 = jnp.sum(jnp.where(sel1, sig, 0.0), axis=-1, keepdims=True)
    b2 = jnp.where(sel1, -jnp.inf, biased)
    m2 = jnp.max(b2, axis=-1, keepdims=True)
    idx2 = jnp.min(jnp.where(b2 == m2, lane, E), axis=-1, keepdims=True)
    sel2 = lane == idx2
    w2 = jnp.sum(jnp.where(sel2, sig, 0.0), axis=-1, keepdims=True)
    denom = w1 + w2 + 1e-20
    cw = (jnp.where(sel1, w1, 0.0) + jnp.where(sel2, w2, 0.0)) / denom * RSF
    lane9 = jax.lax.broadcasted_iota(jnp.int32, (cw.shape[0], E + 1), 1)
    cw_ref[...] = jnp.where(lane9 == E, 1.0, jnp.pad(cw, ((0, 0), (0, 1))))


def _moe_body(x_ref, wg_ref, wu_ref, wd_ref, cw_ref, o_ref):
    e = pl.program_id(1)

    @pl.when(e == 0)
    def _():
        o_ref[...] = jnp.zeros_like(o_ref)

    x = x_ref[...]
    g = jnp.dot(x, wg_ref[0], preferred_element_type=jnp.float32)
    u = jnp.dot(x, wu_ref[0], preferred_element_type=jnp.float32)
    a = g * jax.nn.sigmoid(g) * u
    o = jnp.dot(a, wd_ref[0], preferred_element_type=jnp.float32)
    lane9 = jax.lax.broadcasted_iota(jnp.int32, cw_ref.shape, 1)
    w = jnp.sum(jnp.where(lane9 == e, cw_ref[...], 0.0), axis=-1, keepdims=True)
    o_ref[...] += o * w


def _rms(x, w):
    return x / jnp.sqrt(jnp.mean(x * x, axis=-1, keepdims=True) + EPS) * w


def kernel(positions, hidden_states, residual, in_ln, post_ln, wq, wk, wv, wo,
           qn, kn, gate_w, e_bias, w_gate, w_up, w_down, ws_gate, ws_up,
           ws_down):
    f32 = jnp.float32
    inv = 1.0 / (THETA ** (jnp.arange(0, RD, 2, dtype=f32) / RD))
    ang = positions.astype(f32)[:, None] * inv[None, :]
    cos_t = jnp.cos(ang)
    sin_t = jnp.sin(ang)

    res = hidden_states + residual
    h = _rms(res, in_ln)

    qf, kf, vf = pl.pallas_call(
        _qkv_body,
        grid=(T // BTP,),
        in_specs=[
            pl.BlockSpec((BTP, D), lambda i: (i, 0)),
            pl.BlockSpec((D, H * DH), lambda i: (0, 0)),
            pl.BlockSpec((D, KVH * DH), lambda i: (0, 0)),
            pl.BlockSpec((D, KVH * DH), lambda i: (0, 0)),
        ],
        out_specs=[
            pl.BlockSpec((BTP, H * DH), lambda i: (i, 0)),
            pl.BlockSpec((BTP, KVH * DH), lambda i: (i, 0)),
            pl.BlockSpec((BTP, KVH * DH), lambda i: (i, 0)),
        ],
        out_shape=[
            jax.ShapeDtypeStruct((T, H * DH), f32),
            jax.ShapeDtypeStruct((T, KVH * DH), f32),
            jax.ShapeDtypeStruct((T, KVH * DH), f32),
        ],
    )(h, wq, wk, wv)

    q3 = _rms(qf.reshape(T, H, DH), qn).transpose(1, 0, 2)
    k3 = _rms(kf.reshape(T, KVH, DH), kn).transpose(1, 0, 2)
    v3 = vf.reshape(T, KVH, DH).transpose(1, 0, 2)

    attn = pl.pallas_call(
        _attn_body,
        grid=(H // 2, T // BQ),
        in_specs=[
            pl.BlockSpec((2, BQ, DH), lambda p, qi: (p, qi, 0)),
            pl.BlockSpec((1, T, DH), lambda p, qi: (p // 2, 0, 0)),
            pl.BlockSpec((1, T, DH), lambda p, qi: (p // 2, 0, 0)),
            pl.BlockSpec((BQ, RD // 2), lambda p, qi: (qi, 0)),
            pl.BlockSpec((BQ, RD // 2), lambda p, qi: (qi, 0)),
            pl.BlockSpec((T, RD // 2), lambda p, qi: (0, 0)),
            pl.BlockSpec((T, RD // 2), lambda p, qi: (0, 0)),
        ],
        out_specs=pl.BlockSpec((BQ, 2 * DH), lambda p, qi: (qi, p)),
        out_shape=jax.ShapeDtypeStruct((T, H * DH), f32),
    )(q3, k3, v3, cos_t, sin_t, cos_t, sin_t)

    res2 = pl.pallas_call(
        _postattn_body,
        grid=(T // BT,),
        in_specs=[
            pl.BlockSpec((BT, H * DH), lambda i: (i, 0)),
            pl.BlockSpec((H * DH, D), lambda i: (0, 0)),
            pl.BlockSpec((BT, D), lambda i: (i, 0)),
        ],
        out_specs=pl.BlockSpec((BT, D), lambda i: (i, 0)),
        out_shape=jax.ShapeDtypeStruct((T, D), f32),
    )(attn, wo, res)

    h2 = _rms(res2, post_ln)

    cw9 = pl.pallas_call(
        _router_body,
        grid=(T // BT,),
        in_specs=[
            pl.BlockSpec((BT, D), lambda i: (i, 0)),
            pl.BlockSpec((D, E), lambda i: (0, 0)),
            pl.BlockSpec((1, E), lambda i: (0, 0)),
        ],
        out_specs=pl.BlockSpec((BT, E + 1), lambda i: (i, 0)),
        out_shape=jax.ShapeDtypeStruct((T, E + 1), f32),
    )(h2, gate_w.T, e_bias.reshape(1, E))

    w9g = jnp.concatenate([w_gate, ws_gate[None]], axis=0)
    w9u = jnp.concatenate([w_up, ws_up[None]], axis=0)
    w9d = jnp.concatenate([w_down, ws_down[None]], axis=0)

    out = pl.pallas_call(
        _moe_body,
        grid=(T // BT, E + 1),
        in_specs=[
            pl.BlockSpec((BT, D), lambda i, e: (i, 0)),
            pl.BlockSpec((1, D, I), lambda i, e: (e, 0, 0)),
            pl.BlockSpec((1, D, I), lambda i, e: (e, 0, 0)),
            pl.BlockSpec((1, I, D), lambda i, e: (e, 0, 0)),
            pl.BlockSpec((BT, E + 1), lambda i, e: (i, 0)),
        ],
        out_specs=pl.BlockSpec((BT, D), lambda i, e: (i, 0)),
        out_shape=jax.ShapeDtypeStruct((T, D), f32),
    )(h2, w9g, w9u, w9d, cw9)

    return out, res2
